# baseline (speedup 1.0000x reference)
import functools
import math

import jax
import jax.numpy as jnp
from jax import lax
from jax.experimental import pallas as pl
from jax.experimental.pallas import tpu as pltpu
from jax.experimental.pallas import tpu_sc as plsc

VOCAB_E = 1000000
EMB_E = 64
SCALE = math.sqrt(EMB_E)

_info = plsc.get_sparse_core_info()
NC = _info.num_cores
NS = _info.num_subcores
NW = NC * NS

C = 1024


def _rsqrt_newton(t):
    i = lax.bitcast_convert_type(t, jnp.int32)
    i = jnp.int32(0x5F3759DF) - lax.shift_right_logical(i, 1)
    y = lax.bitcast_convert_type(i, jnp.float32)
    for _ in range(3):
        y = y * (1.5 - 0.5 * t * y * y)
    return y


def _make_kernel(b_total):
    assert b_total % (NW * C) == 0
    b_per_w = b_total // NW
    n_chunks = b_per_w // C
    mesh = plsc.VectorSubcoreMesh(core_axis_name="c", subcore_axis_name="s")

    @functools.partial(
        pl.kernel,
        out_type=jax.ShapeDtypeStruct((b_total, EMB_E), jnp.float32),
        mesh=mesh,
        compiler_params=pltpu.CompilerParams(use_tc_tiling_on_sc=False),
        scratch_types=[
            pltpu.VMEM((C,), jnp.int32),
            pltpu.VMEM((C, EMB_E), jnp.float32),
            pltpu.SemaphoreType.DMA,
        ],
    )
    def kern(tokens_hbm, table_hbm, out_hbm, idx_v, rows_v, sem):
        wid = lax.axis_index("s") * NC + lax.axis_index("c")
        w_base = wid * b_per_w
        lane = lax.iota(jnp.int32, 16)
        perms = [(lane + sh) & 15 for sh in (8, 4, 2, 1)]

        def hsum(v):
            for p in perms:
                v = v + v.at[p].get(mode="promise_in_bounds")
            return v

        def chunk_body(g, carry):
            base = w_base + g * C
            pltpu.sync_copy(tokens_hbm.at[pl.ds(base, C)], idx_v)
            pltpu.async_copy(table_hbm.at[idx_v], rows_v, sem).wait()

            def row_body(r, carry2):
                a0 = rows_v[r, pl.ds(0, 16)]
                a1 = rows_v[r, pl.ds(16, 16)]
                a2 = rows_v[r, pl.ds(32, 16)]
                a3 = rows_v[r, pl.ds(48, 16)]
                s = a0 * a0 + a1 * a1 + a2 * a2 + a3 * a3
                t = hsum(s)
                y = _rsqrt_newton(t)
                scale = SCALE / jnp.maximum(SCALE * t * y, 1e-12)
                rows_v[r, pl.ds(0, 16)] = a0 * scale
                rows_v[r, pl.ds(16, 16)] = a1 * scale
                rows_v[r, pl.ds(32, 16)] = a2 * scale
                rows_v[r, pl.ds(48, 16)] = a3 * scale
                return carry2

            lax.fori_loop(0, C, row_body, 0, unroll=4)
            pltpu.sync_copy(rows_v, out_hbm.at[pl.ds(base, C)])
            return carry

        lax.fori_loop(0, n_chunks, chunk_body, 0)

    return kern


def kernel(tokens, table):
    b_total = tokens.shape[0] * tokens.shape[1]
    flat = tokens.reshape((b_total,)).astype(jnp.int32)
    out = _make_kernel(b_total)(flat, table)
    return out.reshape(tokens.shape + (EMB_E,))
